# baseline (speedup 1.0000x reference)
import functools

import jax
import jax.numpy as jnp
from jax import lax
from jax.experimental import pallas as pl
from jax.experimental.pallas import tpu as pltpu
from jax.experimental.pallas import tpu_sc as plsc

NUM_TYPES = 17
IN_DIM = 128
HID = 128
OUT_DIM = 64
N = 100000

NC, NS, L = 2, 16, 16
NW = NC * NS
CH = 3200
NV = CH // L
RC = 32
NCH = CH // RC
B = 512
LOG2B = 9
NB = 213
NB_PAD = 224
NP = NB * B
HIST = NW * NUM_TYPES * L
LAST = N - (NW - 1) * CH

_mesh = functools.partial(
    plsc.VectorSubcoreMesh, core_axis_name="c", subcore_axis_name="s"
)

_DNUMS = lax.GatherDimensionNumbers(
    offset_dims=(), collapsed_slice_dims=(0,), start_index_map=(0,))


def _vgather(x, idx):
    return lax.gather(x, idx[:, None], _DNUMS, slice_sizes=(1,),
                      mode=lax.GatherScatterMode.PROMISE_IN_BOUNDS)


def _splat_sum(c):
    lanes = lax.iota(jnp.int32, L)
    for d in (1, 2, 4, 8):
        c = c + _vgather(c, jnp.bitwise_xor(lanes, d))
    return c


def _wid():
    return lax.axis_index("s") * NC + lax.axis_index("c")


def _load_types(nt_hbm, tv, w, base):
    @pl.when(w < NW - 1)
    def _():
        pltpu.sync_copy(nt_hbm.at[pl.ds(base, CH)], tv)

    @pl.when(w == NW - 1)
    def _():
        pltpu.sync_copy(nt_hbm.at[pl.ds(base, LAST)], tv.at[pl.ds(0, LAST)])


@functools.partial(
    pl.kernel,
    mesh=_mesh(),
    out_type=jax.ShapeDtypeStruct((HIST,), jnp.int32),
    scratch_types=[
        pltpu.VMEM((CH,), jnp.int32),
        pltpu.VMEM((NUM_TYPES * L,), jnp.int32),
    ],
)
def _sc_hist(nt_hbm, hist_hbm, tv, hv):
    w = _wid()
    base = w * CH
    _load_types(nt_hbm, tv, w, base)

    lanes = lax.iota(jnp.int32, L)
    zvec = jnp.zeros((L,), jnp.int32)
    one = zvec + 1

    def body(v, cnts):
        t = tv[pl.ds(v * L, L)]
        valid = (base + v * L + lanes) < N
        return tuple(
            cnts[i] + jnp.where((t == i) & valid, one, zvec)
            for i in range(NUM_TYPES)
        )

    cnts = lax.fori_loop(0, NV, body, tuple(zvec for _ in range(NUM_TYPES)))
    for i in range(NUM_TYPES):
        hv[pl.ds(i * L, L)] = _splat_sum(cnts[i])
    pltpu.sync_copy(hv, hist_hbm.at[pl.ds(w * NUM_TYPES * L, NUM_TYPES * L)])


@functools.partial(
    pl.kernel,
    mesh=_mesh(),
    out_type=(
        jax.ShapeDtypeStruct((NP + RC, IN_DIM), jnp.float32),
        jax.ShapeDtypeStruct((NW, NCH, RC), jnp.int32),
        jax.ShapeDtypeStruct((NB_PAD,), jnp.int32),
    ),
    scratch_types=[
        pltpu.VMEM((CH,), jnp.int32),
        pltpu.VMEM((HIST,), jnp.int32),
        pltpu.VMEM((NCH, RC), jnp.int32),
        pltpu.VMEM((RC, IN_DIM), jnp.float32),
        pltpu.VMEM((NB_PAD,), jnp.int32),
        pltpu.SemaphoreType.DMA,
    ],
)
def _sc_route(nt_hbm, x_hbm, hist_hbm, xs_hbm, dst_hbm, be_hbm,
              tv, hv, dv, xv, bev, sem):
    w = _wid()
    base = w * CH
    _load_types(nt_hbm, tv, w, base)
    pltpu.sync_copy(hist_hbm, hv)

    lanes = lax.iota(jnp.int32, L)
    zvec = jnp.zeros((L,), jnp.int32)
    one = zvec + 1
    wvec = zvec + w

    ct = [zvec for _ in range(NUM_TYPES)]
    mybase = [zvec for _ in range(NUM_TYPES)]
    for ww in range(NW):
        minev = jnp.where(wvec > ww, one, zvec)
        for i in range(NUM_TYPES):
            hvec = hv[pl.ds((ww * NUM_TYPES + i) * L, L)]
            ct[i] = ct[i] + hvec
            mybase[i] = mybase[i] + hvec * minev

    gstart = []
    g = zvec
    for i in range(NUM_TYPES):
        gstart.append(g)
        padded = ((ct[i] + (B - 1)) >> LOG2B) << LOG2B
        g = g + padded

    wbase = tuple(gstart[i] + mybase[i] for i in range(NUM_TYPES))

    npv = zvec + NP
    lanege = [jnp.where(lanes >= d, one, zvec) for d in range(1, L)]
    shidx = [jnp.maximum(lanes - d, 0) for d in range(1, L)]

    def body(v, wb):
        t = tv[pl.ds(v * L, L)]
        valid = (base + v * L + lanes) < N
        rank = zvec
        for d in range(1, L):
            sh = _vgather(t, shidx[d - 1])
            rank = rank + jnp.where(t == sh, lanege[d - 1], zvec)
        dst = npv
        nwb = []
        for i in range(NUM_TYPES):
            mi = jnp.where((t == i) & valid, one, zvec)
            dst = dst + mi * (wb[i] + rank - dst)
            nwb.append(wb[i] + _splat_sum(mi))
        dv[v // 2, pl.ds((v % 2) * L, L)] = dst
        return tuple(nwb)

    lax.fori_loop(0, NV, body, wbase)

    pltpu.sync_copy(dv, dst_hbm.at[w])

    nch = jnp.minimum(N - base, CH) // RC

    def sbody(j, carry):
        start = pl.multiple_of(base + j * RC, RC)
        pltpu.sync_copy(x_hbm.at[pl.ds(start, RC)], xv)
        pltpu.async_copy(xv, xs_hbm.at[dv.at[j]], sem).wait()
        return carry

    lax.fori_loop(0, nch, sbody, 0)

    @pl.when(w == 0)
    def _():
        gb = [gstart[i] >> LOG2B for i in range(NUM_TYPES)]
        for v in range(NB_PAD // L):
            bvec = v * L + lanes
            be = zvec
            for i in range(1, NUM_TYPES):
                be = jnp.where(bvec >= gb[i], zvec + i, be)
            bev[pl.ds(v * L, L)] = be
        pltpu.sync_copy(bev, be_hbm)


@functools.partial(
    pl.kernel,
    mesh=_mesh(),
    out_type=jax.ShapeDtypeStruct((N, 2 * OUT_DIM), jnp.float32),
    scratch_types=[
        pltpu.VMEM((NCH, RC), jnp.int32),
        pltpu.VMEM((RC, 2 * OUT_DIM), jnp.float32),
        pltpu.SemaphoreType.DMA,
    ],
)
def _sc_unsort(os_hbm, dst_hbm, out_hbm, dv, ov, sem):
    w = _wid()
    base = w * CH
    nrows = jnp.minimum(N - base, CH) // RC
    pltpu.sync_copy(dst_hbm.at[w], dv)

    def gbody(j, carry):
        start = pl.multiple_of(base + j * RC, RC)
        pltpu.async_copy(os_hbm.at[dv.at[j]], ov, sem).wait()
        pltpu.sync_copy(ov, out_hbm.at[pl.ds(start, RC)])
        return carry

    lax.fori_loop(0, nrows, gbody, 0)


def _mlp_body(be_ref, xs_ref, w1_ref, b1_ref, w2_ref, b2_ref, o_ref):
    x = xs_ref[...].astype(jnp.bfloat16)
    h = jnp.dot(x, w1_ref[0], preferred_element_type=jnp.float32)
    h = jax.nn.relu(h + b1_ref[0]).astype(jnp.bfloat16)
    o = jnp.dot(h, w2_ref[0], preferred_element_type=jnp.float32) + b2_ref[0]
    o_ref[...] = jnp.concatenate([o, o], axis=-1)


@jax.jit
def _tc_mlp(be, xs, w1, b1, w2, b2):
    return pl.pallas_call(
        _mlp_body,
        grid_spec=pltpu.PrefetchScalarGridSpec(
            num_scalar_prefetch=1,
            grid=(NB,),
            in_specs=[
                pl.BlockSpec((B, IN_DIM), lambda b, be: (b, 0)),
                pl.BlockSpec((1, IN_DIM, HID), lambda b, be: (be[b], 0, 0)),
                pl.BlockSpec((1, 1, HID), lambda b, be: (be[b], 0, 0)),
                pl.BlockSpec((1, HID, OUT_DIM), lambda b, be: (be[b], 0, 0)),
                pl.BlockSpec((1, 1, OUT_DIM), lambda b, be: (be[b], 0, 0)),
            ],
            out_specs=pl.BlockSpec((B, 2 * OUT_DIM), lambda b, be: (b, 0)),
        ),
        out_shape=jax.ShapeDtypeStruct((NP + RC, 2 * OUT_DIM), jnp.float32),
        compiler_params=pltpu.CompilerParams(
            dimension_semantics=("arbitrary",),
        ),
    )(be, xs, w1, b1, w2, b2)


def kernel(x, node_types, W1, b1, W2, b2):
    hist = _sc_hist(node_types)
    xs, dst, be = _sc_route(node_types, x, hist)
    os_ = _tc_mlp(
        be,
        xs,
        W1.astype(jnp.bfloat16),
        b1.reshape(NUM_TYPES, 1, HID),
        W2.astype(jnp.bfloat16),
        b2.reshape(NUM_TYPES, 1, OUT_DIM),
    )
    out_full = _sc_unsort(os_, dst)
    return out_full[:, :OUT_DIM]
